# Initial kernel scaffold; baseline (speedup 1.0000x reference)
import jax
import jax.numpy as jnp
from jax.experimental import pallas as pl


def kernel(srch, dsth, edge_index, W_in, Wa1, b1, Wa2, b2):
    raise NotImplementedError("write your pallas kernel here")


# speedup vs baseline: 8.5311x; 8.5311x over previous
import functools

import jax
import jax.numpy as jnp
from jax import lax
from jax.experimental import pallas as pl
from jax.experimental.pallas import tpu as pltpu
from jax.experimental.pallas import tpu_sc as plsc

_NC = 2
_NS = 16
_NW = _NC * _NS
_B = 80


def _lrelu(v):
    return jnp.where(v >= 0, v, 0.01 * v)


def _dense_body(x_ref, wt_ref, ws_ref, wd_ref, b1_ref, zp_ref, qb_ref):
    x = x_ref[...]
    z = jnp.dot(x, wt_ref[...], preferred_element_type=jnp.float32)
    p = jnp.dot(z, ws_ref[...], preferred_element_type=jnp.float32)
    q = jnp.dot(z, wd_ref[...], preferred_element_type=jnp.float32) + b1_ref[...]
    zp_ref[...] = jnp.concatenate([z, p], axis=1)
    qb_ref[...] = q


def _dense_stage(srch, w_int, was, wad, b1r, bs):
    n, d = srch.shape
    h = was.shape[1]
    grid = n // bs
    return pl.pallas_call(
        _dense_body,
        grid=(grid,),
        in_specs=[
            pl.BlockSpec((bs, d), lambda i: (i, 0)),
            pl.BlockSpec(w_int.shape, lambda i: (0, 0)),
            pl.BlockSpec(was.shape, lambda i: (0, 0)),
            pl.BlockSpec(wad.shape, lambda i: (0, 0)),
            pl.BlockSpec(b1r.shape, lambda i: (0, 0)),
        ],
        out_specs=[
            pl.BlockSpec((bs, d + h), lambda i: (i, 0)),
            pl.BlockSpec((bs, h), lambda i: (i, 0)),
        ],
        out_shape=[
            jax.ShapeDtypeStruct((n, d + h), jnp.float32),
            jax.ShapeDtypeStruct((n, h), jnp.float32),
        ],
    )(srch, w_int, was, wad, b1r)


def _make_sc_stage(n_pad, e, out_w, acc_w):
    ept = e // _NW
    g = ept // _B
    rows_pt = n_pad // _NS
    mesh = plsc.VectorSubcoreMesh(core_axis_name="c", subcore_axis_name="s")

    @functools.partial(
        pl.kernel,
        mesh=mesh,
        compiler_params=pltpu.CompilerParams(
            use_tc_tiling_on_sc=False, needs_layout_passes=False),
        out_type=jax.ShapeDtypeStruct((_NC * n_pad, acc_w), jnp.float32),
        scratch_types=[
            pltpu.VMEM((_B,), jnp.int32),
            pltpu.VMEM((_B,), jnp.int32),
            pltpu.VMEM((_B, out_w + 64), jnp.float32),
            pltpu.VMEM((_B, 64), jnp.float32),
            pltpu.VMEM((_B, acc_w), jnp.float32),
            pltpu.VMEM((_B,), jnp.float32),
            pltpu.VMEM((64,), jnp.float32),
            pltpu.VMEM((16,), jnp.float32),
            pltpu.VMEM((32, acc_w), jnp.float32),
            pltpu.VMEM_SHARED((n_pad, acc_w), jnp.float32),
            pltpu.SemaphoreType.DMA,
            pltpu.SemaphoreType.DMA,
        ],
    )
    def sc_edges(zp_hbm, qb_hbm, src_hbm, dst_hbm, wa2_hbm, b2_hbm, out_hbm,
                 idx_s, idx_d, zp_rows, qb_rows, mrow, sv, wa2_v, b2_v, zbuf,
                 num_sh, sem1, sem2):
        c = lax.axis_index("c")
        sid = lax.axis_index("s")
        wid = c * _NS + sid

        zero16 = jnp.zeros((16,), jnp.float32)

        def _fill_zero(i, _):
            for j in range(acc_w // 16):
                zbuf[i, pl.ds(16 * j, 16)] = zero16
            return 0

        lax.fori_loop(0, 32, _fill_zero, 0)

        def _zero_acc(k, _):
            pltpu.sync_copy(zbuf, num_sh.at[pl.ds(sid * rows_pt + k * 32, 32)])
            return 0

        lax.fori_loop(0, rows_pt // 32, _zero_acc, 0)

        pltpu.sync_copy(wa2_hbm, wa2_v)
        pltpu.sync_copy(b2_hbm, b2_v)
        plsc.subcore_barrier()

        b2vec = b2_v[...]
        iota16 = lax.iota(jnp.int32, 16)
        zero16f = jnp.zeros((16,), jnp.float32)
        w4 = [wa2_v[pl.ds(16 * i, 16)] for i in range(4)]

        def _batch(gi, _):
            base = wid * ept + gi * _B
            pltpu.sync_copy(src_hbm.at[pl.ds(base, _B)], idx_s)
            pltpu.sync_copy(dst_hbm.at[pl.ds(base, _B)], idx_d)
            cp1 = pltpu.async_copy(zp_hbm.at[idx_s], zp_rows, sem1)
            cp2 = pltpu.async_copy(qb_hbm.at[idx_d], qb_rows, sem2)
            cp1.wait()
            cp2.wait()

            def _dot(gj, _):
                sv_vec = zero16f
                for k in range(16):
                    ei = gj * 16 + k
                    acc = w4[0] * _lrelu(zp_rows[ei, pl.ds(out_w, 16)]
                                         + qb_rows[ei, pl.ds(0, 16)])
                    for i in range(1, 4):
                        acc = acc + w4[i] * _lrelu(
                            zp_rows[ei, pl.ds(out_w + 16 * i, 16)]
                            + qb_rows[ei, pl.ds(16 * i, 16)])
                    sv_vec = jnp.where(iota16 == k, jnp.sum(acc), sv_vec)
                sv[pl.ds(gj * 16, 16)] = jnp.exp(_lrelu(sv_vec + b2vec))
                return 0

            lax.fori_loop(0, _B // 16, _dot, 0)

            def _msg(gj, _):
                s16 = sv[pl.ds(gj * 16, 16)]
                for k in range(16):
                    ei = gj * 16 + k
                    s = s16[k]
                    for i in range(out_w // 16):
                        mrow[ei, pl.ds(16 * i, 16)] = (
                            zp_rows[ei, pl.ds(16 * i, 16)] * s)
                    mrow[ei, pl.ds(out_w, 16)] = jnp.full((16,), s, jnp.float32)
                return 0

            lax.fori_loop(0, _B // 16, _msg, 0)

            pltpu.sync_copy(mrow, num_sh.at[idx_d], add=True)
            return 0

        lax.fori_loop(0, g, _batch, 0)

        plsc.subcore_barrier()

        def _flush(k, _):
            r0 = sid * rows_pt + k * 128
            pltpu.sync_copy(num_sh.at[pl.ds(r0, 128)],
                            out_hbm.at[pl.ds(c * n_pad + r0, 128)])
            return 0

        lax.fori_loop(0, rows_pt // 128, _flush, 0)

    return sc_edges


def _combine_body(parts_ref, out_ref):
    x = parts_ref[...]
    num = x[0, :, :128] + x[1, :, :128]
    den = x[0, :, 128:129] + x[1, :, 128:129]
    out_ref[...] = num / jnp.maximum(den, 1e-16)


def _combine_stage(parts, n, out_w, acc_w, bs):
    grid = n // bs
    return pl.pallas_call(
        _combine_body,
        grid=(grid,),
        in_specs=[pl.BlockSpec((_NC, bs, acc_w), lambda i: (0, i, 0))],
        out_specs=pl.BlockSpec((bs, out_w), lambda i: (i, 0)),
        out_shape=jax.ShapeDtypeStruct((n, out_w), jnp.float32),
    )(parts)


def kernel(srch, dsth, edge_index, W_in, Wa1, b1, Wa2, b2):
    n, d = srch.shape
    out_w = W_in.shape[0]
    h = Wa1.shape[0]
    e = edge_index.shape[1]
    acc_w = out_w + 16

    w_int = W_in.T
    was = Wa1[:, :out_w].T
    wad = Wa1[:, out_w:].T
    b1r = b1.reshape(1, h)

    zp, qb = _dense_stage(srch, w_int, was, wad, b1r, bs=1000)

    src = edge_index[0]
    dst = edge_index[1]
    wa2v = Wa2.reshape(h)
    b2vec = jnp.full((16,), b2[0], jnp.float32)

    n_pad = 10240
    sc = _make_sc_stage(n_pad, e, out_w, acc_w)
    parts = sc(zp, qb, src, dst, wa2v, b2vec)

    hfull = _combine_stage(parts.reshape(_NC, n_pad, acc_w), n_pad, out_w,
                           acc_w, bs=1024)
    return hfull[:n]


# speedup vs baseline: 14.0753x; 1.6499x over previous
import functools

import jax
import jax.numpy as jnp
import numpy as np
from jax import lax
from jax.experimental import pallas as pl
from jax.experimental.pallas import tpu as pltpu
from jax.experimental.pallas import tpu_sc as plsc

_NC = 2
_NS = 16
_NW = _NC * _NS
_B = 80


def _lrelu(v):
    return jnp.where(v >= 0, v, 0.01 * v)


def _ilv_perm(width):
    perm = []
    for g in range(width // 32):
        for p in range(32):
            perm.append(32 * g + (p % 2) * 16 + p // 2)
    return np.array(perm)


def _dense_body(x_ref, wt_ref, ws_ref, wd_ref, b1_ref, zp_ref, qb_ref):
    x = x_ref[...]
    z = jnp.dot(x, wt_ref[...], preferred_element_type=jnp.float32)
    p = jnp.dot(z, ws_ref[...], preferred_element_type=jnp.float32)
    q = jnp.dot(z, wd_ref[...], preferred_element_type=jnp.float32) + b1_ref[...]
    zp_ref[...] = jnp.concatenate([z, p], axis=1).astype(jnp.bfloat16)
    qb_ref[...] = q.astype(jnp.bfloat16)


def _dense_stage(srch, w_int, was, wad, b1r, bs):
    n, d = srch.shape
    h = was.shape[1]
    grid = n // bs
    return pl.pallas_call(
        _dense_body,
        grid=(grid,),
        in_specs=[
            pl.BlockSpec((bs, d), lambda i: (i, 0)),
            pl.BlockSpec(w_int.shape, lambda i: (0, 0)),
            pl.BlockSpec(was.shape, lambda i: (0, 0)),
            pl.BlockSpec(wad.shape, lambda i: (0, 0)),
            pl.BlockSpec(b1r.shape, lambda i: (0, 0)),
        ],
        out_specs=[
            pl.BlockSpec((bs, d + h), lambda i: (i, 0)),
            pl.BlockSpec((bs, h), lambda i: (i, 0)),
        ],
        out_shape=[
            jax.ShapeDtypeStruct((n, d + h), jnp.bfloat16),
            jax.ShapeDtypeStruct((n, h), jnp.bfloat16),
        ],
    )(srch, w_int, was, wad, b1r)


def _make_sc_stage(n_pad, e, out_w, acc_w):
    ept = e // _NW
    g = ept // _B
    rows_pt = n_pad // _NS
    assert g % 2 == 1
    mesh = plsc.VectorSubcoreMesh(core_axis_name="c", subcore_axis_name="s")
    ilv = plsc.PackFormat.INTERLEAVED

    @functools.partial(
        pl.kernel,
        mesh=mesh,
        compiler_params=pltpu.CompilerParams(
            use_tc_tiling_on_sc=False, needs_layout_passes=False),
        out_type=jax.ShapeDtypeStruct((_NC * n_pad, acc_w), jnp.float32),
        scratch_types=[
            pltpu.VMEM((_B,), jnp.int32),
            pltpu.VMEM((_B,), jnp.int32),
            pltpu.VMEM((_B,), jnp.int32),
            pltpu.VMEM((_B,), jnp.int32),
            pltpu.VMEM((_B,), jnp.int32),
            pltpu.VMEM((_B, out_w + 64), jnp.bfloat16),
            pltpu.VMEM((_B, out_w + 64), jnp.bfloat16),
            pltpu.VMEM((_B, 64), jnp.bfloat16),
            pltpu.VMEM((_B, 64), jnp.bfloat16),
            pltpu.VMEM((_B, acc_w), jnp.float32),
            pltpu.VMEM((_B,), jnp.float32),
            pltpu.VMEM((64,), jnp.float32),
            pltpu.VMEM((16,), jnp.float32),
            pltpu.VMEM((8, acc_w), jnp.float32),
            pltpu.VMEM_SHARED((n_pad, acc_w), jnp.float32),
            pltpu.SemaphoreType.DMA,
            pltpu.SemaphoreType.DMA,
            pltpu.SemaphoreType.DMA,
            pltpu.SemaphoreType.DMA,
            pltpu.SemaphoreType.DMA,
            pltpu.SemaphoreType.DMA,
            pltpu.SemaphoreType.DMA,
        ],
    )
    def sc_edges(zp_hbm, qb_hbm, src_hbm, dst_hbm, wa2_hbm, b2_hbm, out_hbm,
                 idx_s0, idx_s1, idx_d0, idx_d1, idx_t,
                 zp_a, zp_b, qb_a, qb_b, mr, sv, wa2_v, b2_v, zbuf,
                 num_sh, semi0, semi1, semz0, semz1, semq0, semq1, semm):
        c = lax.axis_index("c")
        sid = lax.axis_index("s")
        wid = c * _NS + sid

        idx_s = (idx_s0, idx_s1)
        idx_d = (idx_d0, idx_d1)
        zp2 = (zp_a, zp_b)
        qb2 = (qb_a, qb_b)
        semi = (semi0, semi1)
        semz = (semz0, semz1)
        semq = (semq0, semq1)

        zero16 = jnp.zeros((16,), jnp.float32)

        def _fill_zero(i, _):
            for j in range(acc_w // 16):
                zbuf[i, pl.ds(16 * j, 16)] = zero16
            return 0

        lax.fori_loop(0, 8, _fill_zero, 0)

        def _zero_acc(k, _):
            pltpu.sync_copy(zbuf, num_sh.at[pl.ds(sid * rows_pt + k * 8, 8)])
            return 0

        lax.fori_loop(0, rows_pt // 8, _zero_acc, 0)

        pltpu.sync_copy(wa2_hbm, wa2_v)
        pltpu.sync_copy(b2_hbm, b2_v)
        plsc.subcore_barrier()

        b2vec = b2_v[...]
        iota16 = lax.iota(jnp.int32, 16)
        zero16f = jnp.zeros((16,), jnp.float32)
        w4 = [wa2_v[pl.ds(16 * i, 16)] for i in range(4)]

        def _issue_idx(gx, sl):
            base = wid * ept + gx * _B
            pltpu.async_copy(src_hbm.at[pl.ds(base, _B)], idx_s[sl], semi[sl])
            pltpu.async_copy(dst_hbm.at[pl.ds(base, _B)], idx_d[sl], semi[sl])

        def _wait_idx(gx, sl):
            base = wid * ept + gx * _B
            pltpu.make_async_copy(
                src_hbm.at[pl.ds(base, _B)], idx_s[sl], semi[sl]).wait()
            pltpu.make_async_copy(
                dst_hbm.at[pl.ds(base, _B)], idx_d[sl], semi[sl]).wait()

        def _issue_gather(sl):
            pltpu.async_copy(zp_hbm.at[idx_s[sl]], zp2[sl], semz[sl])
            pltpu.async_copy(qb_hbm.at[idx_d[sl]], qb2[sl], semq[sl])

        def _wait_gather(sl):
            pltpu.make_async_copy(zp_hbm.at[idx_s[sl]], zp2[sl], semz[sl]).wait()
            pltpu.make_async_copy(qb_hbm.at[idx_d[sl]], qb2[sl], semq[sl]).wait()

        def _snap_scatter_idx(sl):
            for j in range(_B // 16):
                idx_t[pl.ds(16 * j, 16)] = idx_d[sl][pl.ds(16 * j, 16)]

        def _issue_scatter():
            pltpu.async_copy(mr, num_sh.at[idx_t], semm, add=True)

        def _wait_scatter():
            pltpu.make_async_copy(mr, num_sh.at[idx_t], semm).wait()

        def _compute_a(sl):
            zp_rows = zp2[sl]
            qb_rows = qb2[sl]

            def _dot(gj, _):
                sv_vec = zero16f
                for k in range(16):
                    ei = gj * 16 + k
                    acc = zero16f
                    for i in range(2):
                        pa, pb = plsc.unpack(
                            zp_rows[ei, pl.ds(out_w + 32 * i, 32)], format=ilv)
                        qa, qx = plsc.unpack(
                            qb_rows[ei, pl.ds(32 * i, 32)], format=ilv)
                        acc = acc + w4[2 * i] * _lrelu(pa + qa)
                        acc = acc + w4[2 * i + 1] * _lrelu(pb + qx)
                    sv_vec = jnp.where(iota16 == k, jnp.sum(acc), sv_vec)
                sv[pl.ds(gj * 16, 16)] = jnp.exp(_lrelu(sv_vec + b2vec))
                return 0

            lax.fori_loop(0, _B // 16, _dot, 0)

        def _compute_c(sl):
            zp_rows = zp2[sl]

            def _msg(gj, _):
                s16 = sv[pl.ds(gj * 16, 16)]
                for k in range(16):
                    ei = gj * 16 + k
                    s = s16[k]
                    for i in range(out_w // 32):
                        za, zb = plsc.unpack(
                            zp_rows[ei, pl.ds(32 * i, 32)], format=ilv)
                        mr[ei, pl.ds(32 * i, 16)] = za * s
                        mr[ei, pl.ds(32 * i + 16, 16)] = zb * s
                    mr[ei, pl.ds(out_w, 16)] = jnp.full((16,), s, jnp.float32)
                return 0

            lax.fori_loop(0, _B // 16, _msg, 0)

        npairs = (g - 1) // 2
        _issue_idx(0, 0)
        _issue_idx(1, 1)
        _wait_idx(0, 0)
        _issue_gather(0)

        def _pair(gg, _):
            g0 = 2 * gg
            _wait_gather(0)
            _wait_idx(g0 + 1, 1)
            _issue_gather(1)
            _compute_a(0)

            @pl.when(gg > 0)
            def _():
                _wait_scatter()

            _snap_scatter_idx(0)
            _compute_c(0)
            _issue_scatter()
            _issue_idx(g0 + 2, 0)

            _wait_gather(1)
            _wait_idx(g0 + 2, 0)
            _issue_gather(0)
            _compute_a(1)
            _wait_scatter()
            _snap_scatter_idx(1)
            _compute_c(1)
            _issue_scatter()

            @pl.when(gg < npairs - 1)
            def _():
                _issue_idx(g0 + 3, 1)

            return 0

        lax.fori_loop(0, npairs, _pair, 0)

        _wait_gather(0)
        _compute_a(0)
        _wait_scatter()
        _snap_scatter_idx(0)
        _compute_c(0)
        _issue_scatter()
        _wait_scatter()

        plsc.subcore_barrier()

        def _flush(k, _):
            r0 = sid * rows_pt + k * 128
            pltpu.sync_copy(num_sh.at[pl.ds(r0, 128)],
                            out_hbm.at[pl.ds(c * n_pad + r0, 128)])
            return 0

        lax.fori_loop(0, rows_pt // 128, _flush, 0)

    return sc_edges


def _combine_body(parts_ref, out_ref):
    x = parts_ref[...]
    num = x[0, :, :128] + x[1, :, :128]
    den = x[0, :, 128:129] + x[1, :, 128:129]
    out_ref[...] = num / jnp.maximum(den, 1e-16)


def _combine_stage(parts, n, out_w, acc_w, bs):
    grid = n // bs
    return pl.pallas_call(
        _combine_body,
        grid=(grid,),
        in_specs=[pl.BlockSpec((_NC, bs, acc_w), lambda i: (0, i, 0))],
        out_specs=pl.BlockSpec((bs, out_w), lambda i: (i, 0)),
        out_shape=jax.ShapeDtypeStruct((n, out_w), jnp.float32),
    )(parts)


def kernel(srch, dsth, edge_index, W_in, Wa1, b1, Wa2, b2):
    n, d = srch.shape
    out_w = W_in.shape[0]
    h = Wa1.shape[0]
    e = edge_index.shape[1]
    acc_w = out_w + 16

    permz = _ilv_perm(out_w)
    permh = _ilv_perm(h)

    w_int = W_in.T[:, permz]
    was = Wa1[:, :out_w].T[permz][:, permh]
    wad = Wa1[:, out_w:].T[permz][:, permh]
    b1r = b1[permh].reshape(1, h)

    zp, qb = _dense_stage(srch, w_int, was, wad, b1r, bs=1000)

    src = edge_index[0]
    dst = edge_index[1]
    wa2v = Wa2.reshape(h)
    b2vec = jnp.full((16,), b2[0], jnp.float32)

    n_pad = 10240
    sc = _make_sc_stage(n_pad, e, out_w, acc_w)
    parts = sc(zp, qb, src, dst, wa2v, b2vec)

    hfull = _combine_stage(parts.reshape(_NC, n_pad, acc_w), n_pad, out_w,
                           acc_w, bs=1024)
    return hfull[:n]


# speedup vs baseline: 15.1854x; 1.0789x over previous
import functools

import jax
import jax.numpy as jnp
import numpy as np
from jax import lax
from jax.experimental import pallas as pl
from jax.experimental.pallas import tpu as pltpu
from jax.experimental.pallas import tpu_sc as plsc

_NC = 2
_NS = 16
_NW = _NC * _NS
_B = 80


def _lrelu(v):
    return jnp.where(v >= 0, v, 0.01 * v)


def _ilv_perm(width):
    perm = []
    for g in range(width // 32):
        for p in range(32):
            perm.append(32 * g + (p % 2) * 16 + p // 2)
    return np.array(perm)


def _dense_body(x_ref, wt_ref, ws_ref, wd_ref, b1_ref, zp_ref, qb_ref):
    x = x_ref[...]
    z = jnp.dot(x, wt_ref[...], preferred_element_type=jnp.float32)
    p = jnp.dot(z, ws_ref[...], preferred_element_type=jnp.float32)
    q = jnp.dot(z, wd_ref[...], preferred_element_type=jnp.float32) + b1_ref[...]
    zp_ref[...] = jnp.concatenate([z, p], axis=1).astype(jnp.bfloat16)
    qb_ref[...] = q.astype(jnp.bfloat16)


def _dense_stage(srch, w_int, was, wad, b1r, bs):
    n, d = srch.shape
    h = was.shape[1]
    grid = n // bs
    return pl.pallas_call(
        _dense_body,
        grid=(grid,),
        in_specs=[
            pl.BlockSpec((bs, d), lambda i: (i, 0)),
            pl.BlockSpec(w_int.shape, lambda i: (0, 0)),
            pl.BlockSpec(was.shape, lambda i: (0, 0)),
            pl.BlockSpec(wad.shape, lambda i: (0, 0)),
            pl.BlockSpec(b1r.shape, lambda i: (0, 0)),
        ],
        out_specs=[
            pl.BlockSpec((bs, d + h), lambda i: (i, 0)),
            pl.BlockSpec((bs, h), lambda i: (i, 0)),
        ],
        out_shape=[
            jax.ShapeDtypeStruct((n, d + h), jnp.bfloat16),
            jax.ShapeDtypeStruct((n, h), jnp.bfloat16),
        ],
    )(srch, w_int, was, wad, b1r)


def _make_sc_stage(n_pad, e, out_w, acc_w):
    ept = e // _NW
    g = ept // _B
    rows_pt = n_pad // _NS
    assert g % 2 == 1
    mesh = plsc.VectorSubcoreMesh(core_axis_name="c", subcore_axis_name="s")
    ilv = plsc.PackFormat.INTERLEAVED

    @functools.partial(
        pl.kernel,
        mesh=mesh,
        compiler_params=pltpu.CompilerParams(
            use_tc_tiling_on_sc=False, needs_layout_passes=False),
        out_type=jax.ShapeDtypeStruct((_NC * n_pad, acc_w), jnp.float32),
        scratch_types=[
            pltpu.VMEM((_B,), jnp.int32),
            pltpu.VMEM((_B,), jnp.int32),
            pltpu.VMEM((_B,), jnp.int32),
            pltpu.VMEM((_B,), jnp.int32),
            pltpu.VMEM((_B,), jnp.int32),
            pltpu.VMEM((_B, out_w + 64), jnp.bfloat16),
            pltpu.VMEM((_B, out_w + 64), jnp.bfloat16),
            pltpu.VMEM((_B, 64), jnp.bfloat16),
            pltpu.VMEM((_B, 64), jnp.bfloat16),
            pltpu.VMEM((_B, acc_w), jnp.float32),
            pltpu.VMEM((_B,), jnp.float32),
            pltpu.VMEM((64,), jnp.float32),
            pltpu.VMEM((16,), jnp.float32),
            pltpu.VMEM((8, acc_w), jnp.float32),
            pltpu.VMEM_SHARED((n_pad, acc_w), jnp.float32),
            pltpu.SemaphoreType.DMA,
            pltpu.SemaphoreType.DMA,
            pltpu.SemaphoreType.DMA,
            pltpu.SemaphoreType.DMA,
            pltpu.SemaphoreType.DMA,
            pltpu.SemaphoreType.DMA,
            pltpu.SemaphoreType.DMA,
        ],
    )
    def sc_edges(zp_hbm, qb_hbm, src_hbm, dst_hbm, wa2_hbm, b2_hbm, out_hbm,
                 idx_s0, idx_s1, idx_d0, idx_d1, idx_t,
                 zp_a, zp_b, qb_a, qb_b, mr, sv, wa2_v, b2_v, zbuf,
                 num_sh, semi0, semi1, semz0, semz1, semq0, semq1, semm):
        c = lax.axis_index("c")
        sid = lax.axis_index("s")
        wid = c * _NS + sid

        idx_s = (idx_s0, idx_s1)
        idx_d = (idx_d0, idx_d1)
        zp2 = (zp_a, zp_b)
        qb2 = (qb_a, qb_b)
        semi = (semi0, semi1)
        semz = (semz0, semz1)
        semq = (semq0, semq1)

        zero16 = jnp.zeros((16,), jnp.float32)

        def _fill_zero(i, _):
            for j in range(acc_w // 16):
                zbuf[i, pl.ds(16 * j, 16)] = zero16
            return 0

        lax.fori_loop(0, 8, _fill_zero, 0)

        def _zero_acc(k, _):
            pltpu.sync_copy(zbuf, num_sh.at[pl.ds(sid * rows_pt + k * 8, 8)])
            return 0

        lax.fori_loop(0, rows_pt // 8, _zero_acc, 0)

        pltpu.sync_copy(wa2_hbm, wa2_v)
        pltpu.sync_copy(b2_hbm, b2_v)
        plsc.subcore_barrier()

        b2vec = b2_v[...]
        iota16 = lax.iota(jnp.int32, 16)
        zero16f = jnp.zeros((16,), jnp.float32)
        w4 = [wa2_v[pl.ds(16 * i, 16)] for i in range(4)]

        def _issue_idx(gx, sl):
            base = wid * ept + gx * _B
            pltpu.async_copy(src_hbm.at[pl.ds(base, _B)], idx_s[sl], semi[sl])
            pltpu.async_copy(dst_hbm.at[pl.ds(base, _B)], idx_d[sl], semi[sl])

        def _wait_idx(gx, sl):
            base = wid * ept + gx * _B
            pltpu.make_async_copy(
                src_hbm.at[pl.ds(base, _B)], idx_s[sl], semi[sl]).wait()
            pltpu.make_async_copy(
                dst_hbm.at[pl.ds(base, _B)], idx_d[sl], semi[sl]).wait()

        def _issue_gather(sl):
            pltpu.async_copy(zp_hbm.at[idx_s[sl]], zp2[sl], semz[sl])
            pltpu.async_copy(qb_hbm.at[idx_d[sl]], qb2[sl], semq[sl])

        def _wait_gather(sl):
            pltpu.make_async_copy(zp_hbm.at[idx_s[sl]], zp2[sl], semz[sl]).wait()
            pltpu.make_async_copy(qb_hbm.at[idx_d[sl]], qb2[sl], semq[sl]).wait()

        def _snap_scatter_idx(sl):
            for j in range(_B // 16):
                idx_t[pl.ds(16 * j, 16)] = idx_d[sl][pl.ds(16 * j, 16)]

        def _issue_scatter():
            pltpu.async_copy(mr, num_sh.at[idx_t], semm, add=True)

        def _wait_scatter():
            pltpu.make_async_copy(mr, num_sh.at[idx_t], semm).wait()

        def _compute_a(sl):
            zp_rows = zp2[sl]
            qb_rows = qb2[sl]

            @plsc.parallel_loop(0, _B // 16, 1, unroll=2)
            def _dot(gj):
                sv_vec = zero16f
                for k in range(16):
                    ei = gj * 16 + k
                    acc = zero16f
                    for i in range(2):
                        pa, pb = plsc.unpack(
                            zp_rows[ei, pl.ds(out_w + 32 * i, 32)], format=ilv)
                        qa, qx = plsc.unpack(
                            qb_rows[ei, pl.ds(32 * i, 32)], format=ilv)
                        acc = acc + w4[2 * i] * _lrelu(pa + qa)
                        acc = acc + w4[2 * i + 1] * _lrelu(pb + qx)
                    sv_vec = jnp.where(iota16 == k, jnp.sum(acc), sv_vec)
                sv[pl.ds(gj * 16, 16)] = jnp.exp(_lrelu(sv_vec + b2vec))

        def _compute_c(sl):
            zp_rows = zp2[sl]

            @plsc.parallel_loop(0, _B // 16, 1, unroll=2)
            def _msg(gj):
                s16 = sv[pl.ds(gj * 16, 16)]
                for k in range(16):
                    ei = gj * 16 + k
                    s = s16[k]
                    for i in range(out_w // 32):
                        za, zb = plsc.unpack(
                            zp_rows[ei, pl.ds(32 * i, 32)], format=ilv)
                        mr[ei, pl.ds(32 * i, 16)] = za * s
                        mr[ei, pl.ds(32 * i + 16, 16)] = zb * s
                    mr[ei, pl.ds(out_w, 16)] = jnp.full((16,), s, jnp.float32)

        npairs = (g - 1) // 2
        _issue_idx(0, 0)
        _issue_idx(1, 1)
        _wait_idx(0, 0)
        _issue_gather(0)

        def _pair(gg, _):
            g0 = 2 * gg
            _wait_gather(0)
            _wait_idx(g0 + 1, 1)
            _issue_gather(1)
            _compute_a(0)

            @pl.when(gg > 0)
            def _():
                _wait_scatter()

            _snap_scatter_idx(0)
            _compute_c(0)
            _issue_scatter()
            _issue_idx(g0 + 2, 0)

            _wait_gather(1)
            _wait_idx(g0 + 2, 0)
            _issue_gather(0)
            _compute_a(1)
            _wait_scatter()
            _snap_scatter_idx(1)
            _compute_c(1)
            _issue_scatter()

            @pl.when(gg < npairs - 1)
            def _():
                _issue_idx(g0 + 3, 1)

            return 0

        lax.fori_loop(0, npairs, _pair, 0)

        _wait_gather(0)
        _compute_a(0)
        _wait_scatter()
        _snap_scatter_idx(0)
        _compute_c(0)
        _issue_scatter()
        _wait_scatter()

        plsc.subcore_barrier()

        def _flush(k, _):
            r0 = sid * rows_pt + k * 128
            pltpu.sync_copy(num_sh.at[pl.ds(r0, 128)],
                            out_hbm.at[pl.ds(c * n_pad + r0, 128)])
            return 0

        lax.fori_loop(0, rows_pt // 128, _flush, 0)

    return sc_edges


def _combine_body(parts_ref, out_ref):
    x = parts_ref[...]
    num = x[0, :, :128] + x[1, :, :128]
    den = x[0, :, 128:129] + x[1, :, 128:129]
    out_ref[...] = num / jnp.maximum(den, 1e-16)


def _combine_stage(parts, n, out_w, acc_w, bs):
    grid = n // bs
    return pl.pallas_call(
        _combine_body,
        grid=(grid,),
        in_specs=[pl.BlockSpec((_NC, bs, acc_w), lambda i: (0, i, 0))],
        out_specs=pl.BlockSpec((bs, out_w), lambda i: (i, 0)),
        out_shape=jax.ShapeDtypeStruct((n, out_w), jnp.float32),
    )(parts)


def kernel(srch, dsth, edge_index, W_in, Wa1, b1, Wa2, b2):
    n, d = srch.shape
    out_w = W_in.shape[0]
    h = Wa1.shape[0]
    e = edge_index.shape[1]
    acc_w = out_w + 16

    permz = _ilv_perm(out_w)
    permh = _ilv_perm(h)

    w_int = W_in.T[:, permz]
    was = Wa1[:, :out_w].T[permz][:, permh]
    wad = Wa1[:, out_w:].T[permz][:, permh]
    b1r = b1[permh].reshape(1, h)

    zp, qb = _dense_stage(srch, w_int, was, wad, b1r, bs=1000)

    src = edge_index[0]
    dst = edge_index[1]
    wa2v = Wa2.reshape(h)
    b2vec = jnp.full((16,), b2[0], jnp.float32)

    n_pad = 10240
    sc = _make_sc_stage(n_pad, e, out_w, acc_w)
    parts = sc(zp, qb, src, dst, wa2v, b2vec)

    hfull = _combine_stage(parts.reshape(_NC, n_pad, acc_w), n_pad, out_w,
                           acc_w, bs=1024)
    return hfull[:n]


# speedup vs baseline: 16.8745x; 1.1112x over previous
import functools

import jax
import jax.numpy as jnp
import numpy as np
from jax import lax
from jax.experimental import pallas as pl
from jax.experimental.pallas import tpu as pltpu
from jax.experimental.pallas import tpu_sc as plsc

_NC = 2
_NS = 16
_NW = _NC * _NS
_B = 80


def _lrelu(v):
    return jnp.where(v >= 0, v, 0.01 * v)


def _ilv_perm(width):
    perm = []
    for g in range(width // 32):
        for p in range(32):
            perm.append(32 * g + (p % 2) * 16 + p // 2)
    return np.array(perm)


def _dense_body(x_ref, wt_ref, ws_ref, wd_ref, b1_ref, zp_ref, qb_ref):
    x = x_ref[...]
    z = jnp.dot(x, wt_ref[...], preferred_element_type=jnp.float32)
    p = jnp.dot(z, ws_ref[...], preferred_element_type=jnp.float32)
    q = jnp.dot(z, wd_ref[...], preferred_element_type=jnp.float32) + b1_ref[...]
    zp_ref[...] = jnp.concatenate([z, p], axis=1).astype(jnp.bfloat16)
    qb_ref[...] = q.astype(jnp.bfloat16)


def _dense_stage(srch, w_int, was, wad, b1r, bs):
    n, d = srch.shape
    h = was.shape[1]
    grid = n // bs
    return pl.pallas_call(
        _dense_body,
        grid=(grid,),
        in_specs=[
            pl.BlockSpec((bs, d), lambda i: (i, 0)),
            pl.BlockSpec(w_int.shape, lambda i: (0, 0)),
            pl.BlockSpec(was.shape, lambda i: (0, 0)),
            pl.BlockSpec(wad.shape, lambda i: (0, 0)),
            pl.BlockSpec(b1r.shape, lambda i: (0, 0)),
        ],
        out_specs=[
            pl.BlockSpec((bs, d + h), lambda i: (i, 0)),
            pl.BlockSpec((bs, h), lambda i: (i, 0)),
        ],
        out_shape=[
            jax.ShapeDtypeStruct((n, d + h), jnp.bfloat16),
            jax.ShapeDtypeStruct((n, h), jnp.bfloat16),
        ],
    )(srch, w_int, was, wad, b1r)


def _make_sc_stage(n_pad, e, out_w, acc_w):
    ept = e // _NW
    g = ept // _B
    rows_pt = n_pad // _NS
    assert g % 2 == 1
    mesh = plsc.VectorSubcoreMesh(core_axis_name="c", subcore_axis_name="s")
    ilv = plsc.PackFormat.INTERLEAVED

    @functools.partial(
        pl.kernel,
        mesh=mesh,
        compiler_params=pltpu.CompilerParams(
            use_tc_tiling_on_sc=False, needs_layout_passes=False),
        out_type=jax.ShapeDtypeStruct((_NC * n_pad, acc_w), jnp.float32),
        scratch_types=[
            pltpu.VMEM((_B,), jnp.int32),
            pltpu.VMEM((_B,), jnp.int32),
            pltpu.VMEM((_B,), jnp.int32),
            pltpu.VMEM((_B,), jnp.int32),
            pltpu.VMEM((_B,), jnp.int32),
            pltpu.VMEM((_B, out_w + 64), jnp.bfloat16),
            pltpu.VMEM((_B, out_w + 64), jnp.bfloat16),
            pltpu.VMEM((_B, 64), jnp.bfloat16),
            pltpu.VMEM((_B, 64), jnp.bfloat16),
            pltpu.VMEM((_B, acc_w), jnp.float32),
            pltpu.VMEM((_B,), jnp.float32),
            pltpu.VMEM((64,), jnp.bfloat16),
            pltpu.VMEM((16,), jnp.float32),
            pltpu.VMEM((8, acc_w), jnp.float32),
            pltpu.VMEM_SHARED((n_pad, acc_w), jnp.float32),
            pltpu.SemaphoreType.DMA,
            pltpu.SemaphoreType.DMA,
            pltpu.SemaphoreType.DMA,
            pltpu.SemaphoreType.DMA,
            pltpu.SemaphoreType.DMA,
            pltpu.SemaphoreType.DMA,
            pltpu.SemaphoreType.DMA,
        ],
    )
    def sc_edges(zp_hbm, qb_hbm, src_hbm, dst_hbm, wa2_hbm, b2_hbm, out_hbm,
                 idx_s0, idx_s1, idx_d0, idx_d1, idx_t,
                 zp_a, zp_b, qb_a, qb_b, mr, sv, wa2_v, b2_v, zbuf,
                 num_sh, semi0, semi1, semz0, semz1, semq0, semq1, semm):
        c = lax.axis_index("c")
        sid = lax.axis_index("s")
        wid = c * _NS + sid

        idx_s = (idx_s0, idx_s1)
        idx_d = (idx_d0, idx_d1)
        zp2 = (zp_a, zp_b)
        qb2 = (qb_a, qb_b)
        semi = (semi0, semi1)
        semz = (semz0, semz1)
        semq = (semq0, semq1)

        zero16 = jnp.zeros((16,), jnp.float32)

        def _fill_zero(i, _):
            for j in range(acc_w // 16):
                zbuf[i, pl.ds(16 * j, 16)] = zero16
            return 0

        lax.fori_loop(0, 8, _fill_zero, 0)

        def _zero_acc(k, _):
            pltpu.sync_copy(zbuf, num_sh.at[pl.ds(sid * rows_pt + k * 8, 8)])
            return 0

        lax.fori_loop(0, rows_pt // 8, _zero_acc, 0)

        pltpu.sync_copy(wa2_hbm, wa2_v)
        pltpu.sync_copy(b2_hbm, b2_v)
        plsc.subcore_barrier()

        b2vec = b2_v[...]
        iota16 = lax.iota(jnp.int32, 16)
        zero16f = jnp.zeros((16,), jnp.float32)
        w2b = [wa2_v[pl.ds(32 * i, 32)] for i in range(2)]

        def _issue_idx(gx, sl):
            base = wid * ept + gx * _B
            pltpu.async_copy(src_hbm.at[pl.ds(base, _B)], idx_s[sl], semi[sl])
            pltpu.async_copy(dst_hbm.at[pl.ds(base, _B)], idx_d[sl], semi[sl])

        def _wait_idx(gx, sl):
            base = wid * ept + gx * _B
            pltpu.make_async_copy(
                src_hbm.at[pl.ds(base, _B)], idx_s[sl], semi[sl]).wait()
            pltpu.make_async_copy(
                dst_hbm.at[pl.ds(base, _B)], idx_d[sl], semi[sl]).wait()

        def _issue_gather(sl):
            pltpu.async_copy(zp_hbm.at[idx_s[sl]], zp2[sl], semz[sl])
            pltpu.async_copy(qb_hbm.at[idx_d[sl]], qb2[sl], semq[sl])

        def _wait_gather(sl):
            pltpu.make_async_copy(zp_hbm.at[idx_s[sl]], zp2[sl], semz[sl]).wait()
            pltpu.make_async_copy(qb_hbm.at[idx_d[sl]], qb2[sl], semq[sl]).wait()

        def _snap_scatter_idx(sl):
            for j in range(_B // 16):
                idx_t[pl.ds(16 * j, 16)] = idx_d[sl][pl.ds(16 * j, 16)]

        def _issue_scatter():
            pltpu.async_copy(mr, num_sh.at[idx_t], semm, add=True)

        def _wait_scatter():
            pltpu.make_async_copy(mr, num_sh.at[idx_t], semm).wait()

        def _compute_a(sl):
            zp_rows = zp2[sl]
            qb_rows = qb2[sl]

            @plsc.parallel_loop(0, _B // 16, 1, unroll=2)
            def _dot(gj):
                sv_vec = zero16f
                for k in range(16):
                    ei = gj * 16 + k
                    acc = zero16f
                    for i in range(2):
                        u = _lrelu(zp_rows[ei, pl.ds(out_w + 32 * i, 32)]
                                   + qb_rows[ei, pl.ds(32 * i, 32)]) * w2b[i]
                        ua, ub = plsc.unpack(u, format=ilv)
                        acc = acc + ua + ub
                    sv_vec = jnp.where(iota16 == k, jnp.sum(acc), sv_vec)
                sv[pl.ds(gj * 16, 16)] = jnp.exp(_lrelu(sv_vec + b2vec))

        def _compute_c(sl):
            zp_rows = zp2[sl]

            @plsc.parallel_loop(0, _B // 16, 1, unroll=2)
            def _msg(gj):
                s16 = sv[pl.ds(gj * 16, 16)]
                for k in range(16):
                    ei = gj * 16 + k
                    s = s16[k]
                    for i in range(out_w // 32):
                        za, zb = plsc.unpack(
                            zp_rows[ei, pl.ds(32 * i, 32)], format=ilv)
                        mr[ei, pl.ds(32 * i, 16)] = za * s
                        mr[ei, pl.ds(32 * i + 16, 16)] = zb * s
                    mr[ei, pl.ds(out_w, 16)] = jnp.full((16,), s, jnp.float32)

        npairs = (g - 1) // 2
        _issue_idx(0, 0)
        _issue_idx(1, 1)
        _wait_idx(0, 0)
        _issue_gather(0)

        def _pair(gg, _):
            g0 = 2 * gg
            _wait_gather(0)
            _wait_idx(g0 + 1, 1)
            _issue_gather(1)
            _compute_a(0)

            @pl.when(gg > 0)
            def _():
                _wait_scatter()

            _snap_scatter_idx(0)
            _compute_c(0)
            _issue_scatter()
            _issue_idx(g0 + 2, 0)

            _wait_gather(1)
            _wait_idx(g0 + 2, 0)
            _issue_gather(0)
            _compute_a(1)
            _wait_scatter()
            _snap_scatter_idx(1)
            _compute_c(1)
            _issue_scatter()

            @pl.when(gg < npairs - 1)
            def _():
                _issue_idx(g0 + 3, 1)

            return 0

        lax.fori_loop(0, npairs, _pair, 0)

        _wait_gather(0)
        _compute_a(0)
        _wait_scatter()
        _snap_scatter_idx(0)
        _compute_c(0)
        _issue_scatter()
        _wait_scatter()

        plsc.subcore_barrier()

        def _flush(k, _):
            r0 = sid * rows_pt + k * 128
            pltpu.sync_copy(num_sh.at[pl.ds(r0, 128)],
                            out_hbm.at[pl.ds(c * n_pad + r0, 128)])
            return 0

        lax.fori_loop(0, rows_pt // 128, _flush, 0)

    return sc_edges


def _combine_body(parts_ref, out_ref):
    x = parts_ref[...]
    num = x[0, :, :128] + x[1, :, :128]
    den = x[0, :, 128:129] + x[1, :, 128:129]
    out_ref[...] = num / jnp.maximum(den, 1e-16)


def _combine_stage(parts, n, out_w, acc_w, bs):
    grid = n // bs
    return pl.pallas_call(
        _combine_body,
        grid=(grid,),
        in_specs=[pl.BlockSpec((_NC, bs, acc_w), lambda i: (0, i, 0))],
        out_specs=pl.BlockSpec((bs, out_w), lambda i: (i, 0)),
        out_shape=jax.ShapeDtypeStruct((n, out_w), jnp.float32),
    )(parts)


def kernel(srch, dsth, edge_index, W_in, Wa1, b1, Wa2, b2):
    n, d = srch.shape
    out_w = W_in.shape[0]
    h = Wa1.shape[0]
    e = edge_index.shape[1]
    acc_w = out_w + 16

    permz = _ilv_perm(out_w)
    permh = _ilv_perm(h)

    w_int = W_in.T[:, permz]
    was = Wa1[:, :out_w].T[permz][:, permh]
    wad = Wa1[:, out_w:].T[permz][:, permh]
    b1r = b1[permh].reshape(1, h)

    zp, qb = _dense_stage(srch, w_int, was, wad, b1r, bs=1000)

    src = edge_index[0]
    dst = edge_index[1]
    wa2v = Wa2.reshape(h)[permh].astype(jnp.bfloat16)
    b2vec = jnp.full((16,), b2[0], jnp.float32)

    n_pad = 10240
    sc = _make_sc_stage(n_pad, e, out_w, acc_w)
    parts = sc(zp, qb, src, dst, wa2v, b2vec)

    hfull = _combine_stage(parts.reshape(_NC, n_pad, acc_w), n_pad, out_w,
                           acc_w, bs=1024)
    return hfull[:n]


# speedup vs baseline: 17.0291x; 1.0092x over previous
import functools

import jax
import jax.numpy as jnp
import numpy as np
from jax import lax
from jax.experimental import pallas as pl
from jax.experimental.pallas import tpu as pltpu
from jax.experimental.pallas import tpu_sc as plsc

_NC = 2
_NS = 16
_NW = _NC * _NS
_B = 80


def _lrelu(v):
    return jnp.where(v >= 0, v, 0.01 * v)


def _ilv_perm(width):
    perm = []
    for g in range(width // 32):
        for p in range(32):
            perm.append(32 * g + (p % 2) * 16 + p // 2)
    return np.array(perm)


def _dense_body(x_ref, wt_ref, ws_ref, wd_ref, b1_ref, zp_ref, qb_ref):
    x = x_ref[...]
    z = jnp.dot(x, wt_ref[...], preferred_element_type=jnp.float32)
    p = jnp.dot(z, ws_ref[...], preferred_element_type=jnp.float32)
    q = jnp.dot(z, wd_ref[...], preferred_element_type=jnp.float32) + b1_ref[...]
    zp_ref[...] = jnp.concatenate([z, p], axis=1).astype(jnp.bfloat16)
    qb_ref[...] = q.astype(jnp.bfloat16)


def _dense_stage(srch, w_int, was, wad, b1r, bs):
    n, d = srch.shape
    h = was.shape[1]
    grid = n // bs
    return pl.pallas_call(
        _dense_body,
        grid=(grid,),
        in_specs=[
            pl.BlockSpec((bs, d), lambda i: (i, 0)),
            pl.BlockSpec(w_int.shape, lambda i: (0, 0)),
            pl.BlockSpec(was.shape, lambda i: (0, 0)),
            pl.BlockSpec(wad.shape, lambda i: (0, 0)),
            pl.BlockSpec(b1r.shape, lambda i: (0, 0)),
        ],
        out_specs=[
            pl.BlockSpec((bs, d + h), lambda i: (i, 0)),
            pl.BlockSpec((bs, h), lambda i: (i, 0)),
        ],
        out_shape=[
            jax.ShapeDtypeStruct((n, d + h), jnp.bfloat16),
            jax.ShapeDtypeStruct((n, h), jnp.bfloat16),
        ],
    )(srch, w_int, was, wad, b1r)


def _make_sc_stage(n_pad, e, out_w, acc_w):
    ept = e // _NW
    g = ept // _B
    rows_pt = n_pad // _NS
    assert g % 2 == 1
    mesh = plsc.VectorSubcoreMesh(core_axis_name="c", subcore_axis_name="s")
    ilv = plsc.PackFormat.INTERLEAVED

    @functools.partial(
        pl.kernel,
        mesh=mesh,
        compiler_params=pltpu.CompilerParams(
            use_tc_tiling_on_sc=False, needs_layout_passes=False),
        out_type=jax.ShapeDtypeStruct((_NC * n_pad, acc_w), jnp.float32),
        scratch_types=[
            pltpu.VMEM((_B,), jnp.int32),
            pltpu.VMEM((_B,), jnp.int32),
            pltpu.VMEM((_B,), jnp.int32),
            pltpu.VMEM((_B,), jnp.int32),
            pltpu.VMEM((_B,), jnp.int32),
            pltpu.VMEM((_B, out_w + 64), jnp.bfloat16),
            pltpu.VMEM((_B, out_w + 64), jnp.bfloat16),
            pltpu.VMEM((_B, 64), jnp.bfloat16),
            pltpu.VMEM((_B, 64), jnp.bfloat16),
            pltpu.VMEM((_B, acc_w), jnp.float32),
            pltpu.VMEM((_B,), jnp.float32),
            pltpu.VMEM((64,), jnp.bfloat16),
            pltpu.VMEM((16,), jnp.float32),
            pltpu.VMEM((8, acc_w), jnp.float32),
            pltpu.VMEM_SHARED((n_pad, acc_w), jnp.float32),
            pltpu.SemaphoreType.DMA,
            pltpu.SemaphoreType.DMA,
            pltpu.SemaphoreType.DMA,
            pltpu.SemaphoreType.DMA,
            pltpu.SemaphoreType.DMA,
            pltpu.SemaphoreType.DMA,
            pltpu.SemaphoreType.DMA,
        ],
    )
    def sc_edges(zp_hbm, qb_hbm, src_hbm, dst_hbm, wa2_hbm, b2_hbm, out_hbm,
                 idx_s0, idx_s1, idx_d0, idx_d1, idx_t,
                 zp_a, zp_b, qb_a, qb_b, mr, sv, wa2_v, b2_v, zbuf,
                 num_sh, semi0, semi1, semz0, semz1, semq0, semq1, semm):
        c = lax.axis_index("c")
        sid = lax.axis_index("s")
        wid = c * _NS + sid

        idx_s = (idx_s0, idx_s1)
        idx_d = (idx_d0, idx_d1)
        zp2 = (zp_a, zp_b)
        qb2 = (qb_a, qb_b)
        semi = (semi0, semi1)
        semz = (semz0, semz1)
        semq = (semq0, semq1)

        zero16 = jnp.zeros((16,), jnp.float32)

        def _fill_zero(i, _):
            for j in range(acc_w // 16):
                zbuf[i, pl.ds(16 * j, 16)] = zero16
            return 0

        lax.fori_loop(0, 8, _fill_zero, 0)

        def _zero_acc(k, _):
            pltpu.sync_copy(zbuf, num_sh.at[pl.ds(sid * rows_pt + k * 8, 8)])
            return 0

        lax.fori_loop(0, rows_pt // 8, _zero_acc, 0)

        pltpu.sync_copy(wa2_hbm, wa2_v)
        pltpu.sync_copy(b2_hbm, b2_v)
        plsc.subcore_barrier()

        b2vec = b2_v[...]
        iota16 = lax.iota(jnp.int32, 16)
        zero16f = jnp.zeros((16,), jnp.float32)
        w2b = [wa2_v[pl.ds(32 * i, 32)] for i in range(2)]

        def _issue_idx(gx, sl):
            base = wid * ept + gx * _B
            pltpu.async_copy(src_hbm.at[pl.ds(base, _B)], idx_s[sl], semi[sl])
            pltpu.async_copy(dst_hbm.at[pl.ds(base, _B)], idx_d[sl], semi[sl])

        def _wait_idx(gx, sl):
            base = wid * ept + gx * _B
            pltpu.make_async_copy(
                src_hbm.at[pl.ds(base, _B)], idx_s[sl], semi[sl]).wait()
            pltpu.make_async_copy(
                dst_hbm.at[pl.ds(base, _B)], idx_d[sl], semi[sl]).wait()

        def _issue_gather(sl):
            pltpu.async_copy(zp_hbm.at[idx_s[sl]], zp2[sl], semz[sl])
            pltpu.async_copy(qb_hbm.at[idx_d[sl]], qb2[sl], semq[sl])

        def _wait_gather(sl):
            pltpu.make_async_copy(zp_hbm.at[idx_s[sl]], zp2[sl], semz[sl]).wait()
            pltpu.make_async_copy(qb_hbm.at[idx_d[sl]], qb2[sl], semq[sl]).wait()

        def _snap_scatter_idx(sl):
            for j in range(_B // 16):
                idx_t[pl.ds(16 * j, 16)] = idx_d[sl][pl.ds(16 * j, 16)]

        def _issue_scatter():
            pltpu.async_copy(mr, num_sh.at[idx_t], semm, add=True)

        def _wait_scatter():
            pltpu.make_async_copy(mr, num_sh.at[idx_t], semm).wait()

        def _compute_a(sl):
            zp_rows = zp2[sl]
            qb_rows = qb2[sl]

            @plsc.parallel_loop(0, _B // 16, 1, unroll=4)
            def _dot(gj):
                sv_vec = zero16f
                for k in range(16):
                    ei = gj * 16 + k
                    acc = zero16f
                    for i in range(2):
                        u = _lrelu(zp_rows[ei, pl.ds(out_w + 32 * i, 32)]
                                   + qb_rows[ei, pl.ds(32 * i, 32)]) * w2b[i]
                        ua, ub = plsc.unpack(u, format=ilv)
                        acc = acc + ua + ub
                    sv_vec = jnp.where(iota16 == k, jnp.sum(acc), sv_vec)
                sv[pl.ds(gj * 16, 16)] = jnp.exp(_lrelu(sv_vec + b2vec))

        def _compute_c(sl):
            zp_rows = zp2[sl]

            @plsc.parallel_loop(0, _B // 16, 1, unroll=4)
            def _msg(gj):
                s16 = sv[pl.ds(gj * 16, 16)]
                for k in range(16):
                    ei = gj * 16 + k
                    s = s16[k]
                    for i in range(out_w // 32):
                        za, zb = plsc.unpack(
                            zp_rows[ei, pl.ds(32 * i, 32)], format=ilv)
                        mr[ei, pl.ds(32 * i, 16)] = za * s
                        mr[ei, pl.ds(32 * i + 16, 16)] = zb * s
                    mr[ei, pl.ds(out_w, 16)] = jnp.full((16,), s, jnp.float32)

        npairs = (g - 1) // 2
        _issue_idx(0, 0)
        _issue_idx(1, 1)
        _wait_idx(0, 0)
        _issue_gather(0)

        def _pair(gg, _):
            g0 = 2 * gg
            _wait_gather(0)
            _wait_idx(g0 + 1, 1)
            _issue_gather(1)
            _compute_a(0)

            @pl.when(gg > 0)
            def _():
                _wait_scatter()

            _snap_scatter_idx(0)
            _compute_c(0)
            _issue_scatter()
            _issue_idx(g0 + 2, 0)

            _wait_gather(1)
            _wait_idx(g0 + 2, 0)
            _issue_gather(0)
            _compute_a(1)
            _wait_scatter()
            _snap_scatter_idx(1)
            _compute_c(1)
            _issue_scatter()

            @pl.when(gg < npairs - 1)
            def _():
                _issue_idx(g0 + 3, 1)

            return 0

        lax.fori_loop(0, npairs, _pair, 0)

        _wait_gather(0)
        _compute_a(0)
        _wait_scatter()
        _snap_scatter_idx(0)
        _compute_c(0)
        _issue_scatter()
        _wait_scatter()

        plsc.subcore_barrier()

        def _flush(k, _):
            r0 = sid * rows_pt + k * 128
            pltpu.sync_copy(num_sh.at[pl.ds(r0, 128)],
                            out_hbm.at[pl.ds(c * n_pad + r0, 128)])
            return 0

        lax.fori_loop(0, rows_pt // 128, _flush, 0)

    return sc_edges


def _combine_body(parts_ref, out_ref):
    x = parts_ref[...]
    num = x[0, :, :128] + x[1, :, :128]
    den = x[0, :, 128:129] + x[1, :, 128:129]
    out_ref[...] = num / jnp.maximum(den, 1e-16)


def _combine_stage(parts, n, out_w, acc_w, bs):
    grid = n // bs
    return pl.pallas_call(
        _combine_body,
        grid=(grid,),
        in_specs=[pl.BlockSpec((_NC, bs, acc_w), lambda i: (0, i, 0))],
        out_specs=pl.BlockSpec((bs, out_w), lambda i: (i, 0)),
        out_shape=jax.ShapeDtypeStruct((n, out_w), jnp.float32),
    )(parts)


def kernel(srch, dsth, edge_index, W_in, Wa1, b1, Wa2, b2):
    n, d = srch.shape
    out_w = W_in.shape[0]
    h = Wa1.shape[0]
    e = edge_index.shape[1]
    acc_w = out_w + 16

    permz = _ilv_perm(out_w)
    permh = _ilv_perm(h)

    w_int = W_in.T[:, permz]
    was = Wa1[:, :out_w].T[permz][:, permh]
    wad = Wa1[:, out_w:].T[permz][:, permh]
    b1r = b1[permh].reshape(1, h)

    zp, qb = _dense_stage(srch, w_int, was, wad, b1r, bs=1000)

    src = edge_index[0]
    dst = edge_index[1]
    wa2v = Wa2.reshape(h)[permh].astype(jnp.bfloat16)
    b2vec = jnp.full((16,), b2[0], jnp.float32)

    n_pad = 10240
    sc = _make_sc_stage(n_pad, e, out_w, acc_w)
    parts = sc(zp, qb, src, dst, wa2v, b2vec)

    hfull = _combine_stage(parts.reshape(_NC, n_pad, acc_w), n_pad, out_w,
                           acc_w, bs=1024)
    return hfull[:n]
